# baseline (device time: 47962 ns/iter reference)
import jax
import jax.numpy as jnp
from jax import lax
from jax.experimental import pallas as pl
from jax.experimental.pallas import tpu as pltpu

N_DEV = 4
SEG = 4


def kernel(x, W1, W2):
    m_per, d = x.shape
    mh = m_per // 2
    sr = mh // SEG

    def body(x_ref, w1_ref, w2_ref, out_ref,
             xrA, pA, rsA, xrB, pB, rsB,
             agA_s, agA_r, rsA_s, rsA_r,
             agB_s, agB_r, rsB_s, rsB_r):
        my = lax.axis_index("i")
        left = lax.rem(my + N_DEV - 1, N_DEV)
        right = lax.rem(my + 1, N_DEV)

        barrier = pltpu.get_barrier_semaphore()
        for nbr in (left, right):
            pl.semaphore_signal(barrier, inc=1, device_id=(nbr,),
                                device_id_type=pl.DeviceIdType.MESH)
        pl.semaphore_wait(barrier, 2)

        rings = (
            dict(xr=xrA, p=pA, rs=rsA, ag_s=agA_s, ag_r=agA_r,
                 rs_s=rsA_s, rs_r=rsA_r, dst=right, off=0),
            dict(xr=xrB, p=pB, rs=rsB, ag_s=agB_s, ag_r=agB_r,
                 rs_s=rsB_s, rs_r=rsB_r, dst=left, off=mh),
        )

        def rows(g):
            return slice(g * sr, (g + 1) * sr)

        def ag_rdma(r, h, g):
            return pltpu.make_async_remote_copy(
                src_ref=r["xr"].at[h, rows(g), :],
                dst_ref=r["xr"].at[h + 1, rows(g), :],
                send_sem=r["ag_s"].at[h, g],
                recv_sem=r["ag_r"].at[h, g],
                device_id=(r["dst"],), device_id_type=pl.DeviceIdType.MESH)

        def rs_rdma(r, s, g):
            return pltpu.make_async_remote_copy(
                src_ref=r["p"].at[s, rows(g), :],
                dst_ref=r["rs"].at[s, rows(g), :],
                send_sem=r["rs_s"].at[s, g],
                recv_sem=r["rs_r"].at[s, g],
                device_id=(r["dst"],), device_id_type=pl.DeviceIdType.MESH)

        def f(r, j, g):
            xc = r["xr"][j, rows(g), :]
            h1 = jnp.dot(xc, w1_ref[:, :],
                         preferred_element_type=jnp.float32)
            h1 = h1 * (1.0 / (1.0 + jnp.exp(-h1)))
            return jnp.dot(h1, w2_ref[:, :],
                           preferred_element_type=jnp.float32)

        sends = []

        def start(desc):
            desc.start()
            sends.append(desc)
            return desc

        ag = {}
        rs = {}

        for ri, r in enumerate(rings):
            r["xr"][0] = x_ref[r["off"]:r["off"] + mh, :]
        for g in range(SEG):
            for ri, r in enumerate(rings):
                ag[ri, 0, g] = start(ag_rdma(r, 0, g))
        for g in range(SEG):
            for ri, r in enumerate(rings):
                r["p"][3, rows(g), :] = f(r, 0, g)

        for g in range(SEG):
            for ri, r in enumerate(rings):
                ag[ri, 0, g].wait_recv()
                ag[ri, 1, g] = start(ag_rdma(r, 1, g))
                r["p"][0, rows(g), :] = f(r, 1, g)
                rs[ri, 0, g] = start(rs_rdma(r, 0, g))

        for g in range(SEG):
            for ri, r in enumerate(rings):
                ag[ri, 1, g].wait_recv()
                ag[ri, 2, g] = start(ag_rdma(r, 2, g))
                r["p"][1, rows(g), :] = f(r, 2, g)

        for g in range(SEG):
            for ri, r in enumerate(rings):
                rs[ri, 0, g].wait_recv()
                r["p"][1, rows(g), :] = r["p"][1, rows(g), :] + r["rs"][0, rows(g), :]
                rs[ri, 1, g] = start(rs_rdma(r, 1, g))

        for g in range(SEG):
            for ri, r in enumerate(rings):
                ag[ri, 2, g].wait_recv()
                r["p"][2, rows(g), :] = f(r, 3, g)

        for g in range(SEG):
            for ri, r in enumerate(rings):
                rs[ri, 1, g].wait_recv()
                r["p"][2, rows(g), :] = r["p"][2, rows(g), :] + r["rs"][1, rows(g), :]
                rs[ri, 2, g] = start(rs_rdma(r, 2, g))

        for g in range(SEG):
            for ri, r in enumerate(rings):
                rs[ri, 2, g].wait_recv()
                o = slice(r["off"] + g * sr, r["off"] + (g + 1) * sr)
                out_ref[o, :] = r["p"][3, rows(g), :] + r["rs"][2, rows(g), :]

        for desc in sends:
            desc.wait_send()

    half = (N_DEV, mh, d)
    rs_shape = (N_DEV - 1, mh, d)
    sem2 = pltpu.SemaphoreType.DMA((N_DEV - 1, SEG))
    return pl.pallas_call(
        body,
        out_shape=jax.ShapeDtypeStruct((m_per, d), jnp.float32),
        in_specs=[pl.BlockSpec(memory_space=pltpu.VMEM)] * 3,
        out_specs=pl.BlockSpec(memory_space=pltpu.VMEM),
        scratch_shapes=[
            pltpu.VMEM(half, jnp.float32),
            pltpu.VMEM(half, jnp.float32),
            pltpu.VMEM(rs_shape, jnp.float32),
            pltpu.VMEM(half, jnp.float32),
            pltpu.VMEM(half, jnp.float32),
            pltpu.VMEM(rs_shape, jnp.float32),
            sem2, sem2, sem2, sem2,
            sem2, sem2, sem2, sem2,
        ],
        compiler_params=pltpu.CompilerParams(collective_id=0),
    )(x, W1, W2)


# device time: 45686 ns/iter; 1.0498x vs baseline; 1.0498x over previous
import jax
import jax.numpy as jnp
from jax import lax
from jax.experimental import pallas as pl
from jax.experimental.pallas import tpu as pltpu

N_DEV = 4
SEG = 2


def kernel(x, W1, W2):
    m_per, d = x.shape
    mh = m_per // 2
    sr = mh // SEG

    def body(x_ref, w1_ref, w2_ref, out_ref,
             xrA, pA, rsA, xrB, pB, rsB,
             agA_s, agA_r, rsA_s, rsA_r,
             agB_s, agB_r, rsB_s, rsB_r):
        my = lax.axis_index("i")
        left = lax.rem(my + N_DEV - 1, N_DEV)
        right = lax.rem(my + 1, N_DEV)

        barrier = pltpu.get_barrier_semaphore()
        for nbr in (left, right):
            pl.semaphore_signal(barrier, inc=1, device_id=(nbr,),
                                device_id_type=pl.DeviceIdType.MESH)
        pl.semaphore_wait(barrier, 2)

        rings = (
            dict(xr=xrA, p=pA, rs=rsA, ag_s=agA_s, ag_r=agA_r,
                 rs_s=rsA_s, rs_r=rsA_r, dst=right, off=0),
            dict(xr=xrB, p=pB, rs=rsB, ag_s=agB_s, ag_r=agB_r,
                 rs_s=rsB_s, rs_r=rsB_r, dst=left, off=mh),
        )

        def rows(g):
            return slice(g * sr, (g + 1) * sr)

        def ag_rdma(r, h, g):
            if h == 0:
                src = x_ref.at[r["off"] + g * sr:r["off"] + (g + 1) * sr, :]
            else:
                src = r["xr"].at[h, rows(g), :]
            return pltpu.make_async_remote_copy(
                src_ref=src,
                dst_ref=r["xr"].at[h + 1, rows(g), :],
                send_sem=r["ag_s"].at[h, g],
                recv_sem=r["ag_r"].at[h, g],
                device_id=(r["dst"],), device_id_type=pl.DeviceIdType.MESH)

        def rs_rdma(r, s, g):
            return pltpu.make_async_remote_copy(
                src_ref=r["p"].at[s, rows(g), :],
                dst_ref=r["rs"].at[s, rows(g), :],
                send_sem=r["rs_s"].at[s, g],
                recv_sem=r["rs_r"].at[s, g],
                device_id=(r["dst"],), device_id_type=pl.DeviceIdType.MESH)

        def f(r, j, g):
            if j == 0:
                xc = x_ref[r["off"] + g * sr:r["off"] + (g + 1) * sr, :]
            else:
                xc = r["xr"][j, rows(g), :]
            h1 = jnp.dot(xc, w1_ref[:, :],
                         preferred_element_type=jnp.float32)
            h1 = h1 * (1.0 / (1.0 + jnp.exp(-h1)))
            return jnp.dot(h1, w2_ref[:, :],
                           preferred_element_type=jnp.float32)

        sends = []

        def start(desc):
            desc.start()
            sends.append(desc)
            return desc

        ag = {}
        rs = {}

        for g in range(SEG):
            for ri, r in enumerate(rings):
                ag[ri, 0, g] = start(ag_rdma(r, 0, g))
        for g in range(SEG):
            for ri, r in enumerate(rings):
                r["p"][3, rows(g), :] = f(r, 0, g)

        for g in range(SEG):
            for ri, r in enumerate(rings):
                ag[ri, 0, g].wait_recv()
                ag[ri, 1, g] = start(ag_rdma(r, 1, g))
                r["p"][0, rows(g), :] = f(r, 1, g)
                rs[ri, 0, g] = start(rs_rdma(r, 0, g))

        for g in range(SEG):
            for ri, r in enumerate(rings):
                ag[ri, 1, g].wait_recv()
                ag[ri, 2, g] = start(ag_rdma(r, 2, g))
                r["p"][1, rows(g), :] = f(r, 2, g)

        for g in range(SEG):
            for ri, r in enumerate(rings):
                rs[ri, 0, g].wait_recv()
                r["p"][1, rows(g), :] = r["p"][1, rows(g), :] + r["rs"][0, rows(g), :]
                rs[ri, 1, g] = start(rs_rdma(r, 1, g))

        for g in range(SEG):
            for ri, r in enumerate(rings):
                ag[ri, 2, g].wait_recv()
                r["p"][2, rows(g), :] = f(r, 3, g)

        for g in range(SEG):
            for ri, r in enumerate(rings):
                rs[ri, 1, g].wait_recv()
                r["p"][2, rows(g), :] = r["p"][2, rows(g), :] + r["rs"][1, rows(g), :]
                rs[ri, 2, g] = start(rs_rdma(r, 2, g))

        for g in range(SEG):
            for ri, r in enumerate(rings):
                rs[ri, 2, g].wait_recv()
                o = slice(r["off"] + g * sr, r["off"] + (g + 1) * sr)
                out_ref[o, :] = r["p"][3, rows(g), :] + r["rs"][2, rows(g), :]

        for desc in sends:
            desc.wait_send()

    half = (N_DEV, mh, d)
    rs_shape = (N_DEV - 1, mh, d)
    sem2 = pltpu.SemaphoreType.DMA((N_DEV - 1, SEG))
    return pl.pallas_call(
        body,
        out_shape=jax.ShapeDtypeStruct((m_per, d), jnp.float32),
        in_specs=[pl.BlockSpec(memory_space=pltpu.VMEM)] * 3,
        out_specs=pl.BlockSpec(memory_space=pltpu.VMEM),
        scratch_shapes=[
            pltpu.VMEM(half, jnp.float32),
            pltpu.VMEM(half, jnp.float32),
            pltpu.VMEM(rs_shape, jnp.float32),
            pltpu.VMEM(half, jnp.float32),
            pltpu.VMEM(half, jnp.float32),
            pltpu.VMEM(rs_shape, jnp.float32),
            sem2, sem2, sem2, sem2,
            sem2, sem2, sem2, sem2,
        ],
        compiler_params=pltpu.CompilerParams(collective_id=0),
    )(x, W1, W2)


# device time: 45672 ns/iter; 1.0501x vs baseline; 1.0003x over previous
import jax
import jax.numpy as jnp
from jax import lax
from jax.experimental import pallas as pl
from jax.experimental.pallas import tpu as pltpu

N_DEV = 4
SEG = 2
L, R, O = 0, 1, 2
COMB, EARLY, DIRECT = 0, 1, 2


def kernel(x, W1, W2):
    m_per, d = x.shape
    mh = m_per // 2
    sr = mh // SEG

    def body(x_ref, w1_ref, w2_ref, out_ref,
             xgA, pA, rinA, pownA, xgB, pB, rinB, pownB,
             agSA, agRA, rsSA, rsRA, agSB, agRB, rsSB, rsRB):
        my = lax.axis_index("i")
        left = lax.rem(my + N_DEV - 1, N_DEV)
        right = lax.rem(my + 1, N_DEV)

        barrier = pltpu.get_barrier_semaphore()
        for nbr in (left, right):
            pl.semaphore_signal(barrier, inc=1, device_id=(nbr,),
                                device_id_type=pl.DeviceIdType.MESH)
        pl.semaphore_wait(barrier, 2)

        rings = (
            dict(xg=xgA, p=pA, rin=rinA, pown=pownA, agS=agSA, agR=agRA,
                 rsS=rsSA, rsR=rsRA, fwd=right, bwd=left, off=0),
            dict(xg=xgB, p=pB, rin=rinB, pown=pownB, agS=agSB, agR=agRB,
                 rsS=rsSB, rsR=rsRB, fwd=left, bwd=right, off=mh),
        )

        def rows(g):
            return slice(g * sr, (g + 1) * sr)

        def xrows(r, g):
            return slice(r["off"] + g * sr, r["off"] + (g + 1) * sr)

        def rdma(src, dst, ssem, rsem, dev):
            return pltpu.make_async_remote_copy(
                src_ref=src, dst_ref=dst, send_sem=ssem, recv_sem=rsem,
                device_id=(dev,), device_id_type=pl.DeviceIdType.MESH)

        def f(xc):
            h1 = jnp.dot(xc, w1_ref[:, :],
                         preferred_element_type=jnp.float32)
            h1 = h1 * (1.0 / (1.0 + jnp.exp(-h1)))
            return jnp.dot(h1, w2_ref[:, :],
                           preferred_element_type=jnp.float32)

        sends = []

        def start(desc):
            desc.start()
            sends.append(desc)
            return desc

        own_f, own_b, fwdd, early, comb, direct = {}, {}, {}, {}, {}, {}

        for g in range(SEG):
            for ri, r in enumerate(rings):
                own_f[ri, g] = start(rdma(
                    x_ref.at[xrows(r, g), :], r["xg"].at[L, rows(g), :],
                    r["agS"].at[L, g], r["agR"].at[L, g], r["fwd"]))
                own_b[ri, g] = start(rdma(
                    x_ref.at[xrows(r, g), :], r["xg"].at[R, rows(g), :],
                    r["agS"].at[R, g], r["agR"].at[R, g], r["bwd"]))
        for g in range(SEG):
            for ri, r in enumerate(rings):
                r["pown"][rows(g), :] = f(x_ref[xrows(r, g), :])

        for g in range(SEG):
            for ri, r in enumerate(rings):
                own_f[ri, g].wait_recv()
                fwdd[ri, g] = start(rdma(
                    r["xg"].at[L, rows(g), :], r["xg"].at[O, rows(g), :],
                    r["agS"].at[O, g], r["agR"].at[O, g], r["fwd"]))
                r["p"][DIRECT, rows(g), :] = f(r["xg"][L, rows(g), :])
                direct[ri, g] = start(rdma(
                    r["p"].at[DIRECT, rows(g), :],
                    r["rin"].at[2, rows(g), :],
                    r["rsS"].at[2, g], r["rsR"].at[2, g], r["bwd"]))

        for g in range(SEG):
            for ri, r in enumerate(rings):
                own_b[ri, g].wait_recv()
                r["p"][COMB, rows(g), :] = f(r["xg"][R, rows(g), :])

        for g in range(SEG):
            for ri, r in enumerate(rings):
                fwdd[ri, g].wait_recv()
                r["p"][EARLY, rows(g), :] = f(r["xg"][O, rows(g), :])
                early[ri, g] = start(rdma(
                    r["p"].at[EARLY, rows(g), :],
                    r["rin"].at[0, rows(g), :],
                    r["rsS"].at[0, g], r["rsR"].at[0, g], r["fwd"]))

        for g in range(SEG):
            for ri, r in enumerate(rings):
                early[ri, g].wait_recv()
                r["p"][COMB, rows(g), :] = (
                    r["p"][COMB, rows(g), :] + r["rin"][0, rows(g), :])
                comb[ri, g] = start(rdma(
                    r["p"].at[COMB, rows(g), :],
                    r["rin"].at[1, rows(g), :],
                    r["rsS"].at[1, g], r["rsR"].at[1, g], r["fwd"]))

        for g in range(SEG):
            for ri, r in enumerate(rings):
                direct[ri, g].wait_recv()
                comb[ri, g].wait_recv()
                out_ref[xrows(r, g), :] = (
                    r["pown"][rows(g), :] + r["rin"][1, rows(g), :]
                    + r["rin"][2, rows(g), :])

        for desc in sends:
            desc.wait_send()

    buf3 = (3, mh, d)
    sem3 = pltpu.SemaphoreType.DMA((3, SEG))
    return pl.pallas_call(
        body,
        out_shape=jax.ShapeDtypeStruct((m_per, d), jnp.float32),
        in_specs=[pl.BlockSpec(memory_space=pltpu.VMEM)] * 3,
        out_specs=pl.BlockSpec(memory_space=pltpu.VMEM),
        scratch_shapes=[
            pltpu.VMEM(buf3, jnp.float32),
            pltpu.VMEM(buf3, jnp.float32),
            pltpu.VMEM(buf3, jnp.float32),
            pltpu.VMEM((mh, d), jnp.float32),
            pltpu.VMEM(buf3, jnp.float32),
            pltpu.VMEM(buf3, jnp.float32),
            pltpu.VMEM(buf3, jnp.float32),
            pltpu.VMEM((mh, d), jnp.float32),
            sem3, sem3, sem3, sem3,
            sem3, sem3, sem3, sem3,
        ],
        compiler_params=pltpu.CompilerParams(collective_id=0),
    )(x, W1, W2)
